# Initial kernel scaffold; baseline (speedup 1.0000x reference)
import jax
import jax.numpy as jnp
from jax.experimental import pallas as pl


def kernel(queries, features):
    raise NotImplementedError("write your pallas kernel here")


# speedup vs baseline: 3.6695x; 3.6695x over previous
import functools

import jax
import jax.numpy as jnp
from jax import lax
from jax.experimental import pallas as pl
from jax.experimental.pallas import tpu as pltpu
from jax.experimental.pallas import tpu_sc as plsc

QN = 1024
KN = 65536
DN = 512
TK = 8
CHUNK = 128
NCHUNK = KN // CHUNK
BK = 2048
NBLK = KN // BK
CPB = BK // CHUNK


def _matmul_body(q_ref, f_ref, sim_ref, cmax_ref):
    s = lax.dot_general(
        q_ref[...], f_ref[...], (((1,), (1,)), ((), ())),
        preferred_element_type=jnp.float32)
    sim_ref[...] = s
    cmax_ref[...] = jnp.max(s.reshape(QN, CPB, CHUNK), axis=2)[None]


def _sim_chunkmax(queries, features):
    return pl.pallas_call(
        _matmul_body,
        grid=(NBLK,),
        in_specs=[
            pl.BlockSpec((QN, DN), lambda k: (0, 0)),
            pl.BlockSpec((BK, DN), lambda k: (k, 0)),
        ],
        out_specs=[
            pl.BlockSpec((QN, BK), lambda k: (0, k)),
            pl.BlockSpec((1, QN, CPB), lambda k: (k, 0, 0)),
        ],
        out_shape=[
            jax.ShapeDtypeStruct((QN, KN), jnp.float32),
            jax.ShapeDtypeStruct((NBLK, QN, CPB), jnp.float32),
        ],
    )(queries, features)


def _chunk_topk_body(cmax_ref, out_ref):
    vals = cmax_ref[...]
    lane = lax.broadcasted_iota(jnp.int32, (QN, NCHUNK), 1)
    row = lax.broadcasted_iota(jnp.int32, (QN, 1), 0)
    neg = jnp.float32(-jnp.inf)
    cols = []
    for _ in range(TK):
        m = jnp.max(vals, axis=1, keepdims=True)
        am = jnp.min(jnp.where(vals == m, lane, NCHUNK), axis=1, keepdims=True)
        cols.append(row * NCHUNK + am)
        vals = jnp.where(lane == am, neg, vals)
    out_ref[...] = jnp.concatenate(cols, axis=1)


def _chunk_topk(cmax):
    return pl.pallas_call(
        _chunk_topk_body,
        out_shape=jax.ShapeDtypeStruct((QN, TK), jnp.int32),
    )(cmax)


def _final_topk_body(cand_ref, cid_ref, out_ref):
    vals = cand_ref[...]
    row = lax.broadcasted_iota(jnp.int32, (QN, 1), 0)
    c = cid_ref[...] - row * NCHUNK
    lane128 = lax.broadcasted_iota(jnp.int32, (QN, TK, CHUNK), 2)
    gidx = (c.reshape(QN, TK, 1) * CHUNK + lane128).reshape(QN, TK * CHUNK)
    lane = lax.broadcasted_iota(jnp.int32, (QN, TK * CHUNK), 1)
    neg = jnp.float32(-jnp.inf)
    outs = []
    for _ in range(TK):
        m = jnp.max(vals, axis=1, keepdims=True)
        am = jnp.min(jnp.where(vals == m, lane, TK * CHUNK),
                     axis=1, keepdims=True)
        sel = lane == am
        outs.append(jnp.max(jnp.where(sel, gidx, -1), axis=1, keepdims=True))
        vals = jnp.where(sel, neg, vals)
    out_ref[...] = jnp.concatenate(outs, axis=1)


def _final_topk(cand, cids):
    return pl.pallas_call(
        _final_topk_body,
        out_shape=jax.ShapeDtypeStruct((QN, TK), jnp.int32),
    )(cand, cids)


_NC = 2
_NS = 16
_NW = _NC * _NS


def _sc_gather_cand(sim2, idx2):
    mesh = plsc.VectorSubcoreMesh(core_axis_name="c", subcore_axis_name="s")

    @functools.partial(
        pl.kernel, mesh=mesh,
        out_type=jax.ShapeDtypeStruct((QN * TK, CHUNK), jnp.float32),
        scratch_types=[
            pltpu.VMEM((2, 128), jnp.int32),
            pltpu.VMEM((256, CHUNK), jnp.float32),
            pltpu.SemaphoreType.DMA,
        ],
    )
    def k(sim_hbm, idx_hbm, out_hbm, idx_v, rows_v, sem):
        wid = lax.axis_index("s") * _NC + lax.axis_index("c")
        pltpu.sync_copy(idx_hbm.at[pl.ds(wid * 2, 2)], idx_v)
        for j in range(2):
            pltpu.async_copy(sim_hbm.at[idx_v.at[j]],
                             rows_v.at[pl.ds(j * 128, 128)], sem).wait()
        pltpu.sync_copy(rows_v, out_hbm.at[pl.ds(wid * 256, 256)])

    return k(sim2, idx2)


def _sc_gather_feats(features, idx2):
    mesh = plsc.VectorSubcoreMesh(core_axis_name="c", subcore_axis_name="s")

    @functools.partial(
        pl.kernel, mesh=mesh,
        out_type=jax.ShapeDtypeStruct((QN * TK, DN), jnp.float32),
        scratch_types=[
            pltpu.VMEM((2, 128), jnp.int32),
            pltpu.VMEM((128, DN), jnp.float32),
            pltpu.SemaphoreType.DMA,
        ],
    )
    def k(f_hbm, idx_hbm, out_hbm, idx_v, rows_v, sem):
        wid = lax.axis_index("s") * _NC + lax.axis_index("c")
        pltpu.sync_copy(idx_hbm.at[pl.ds(wid * 2, 2)], idx_v)
        for j in range(2):
            pltpu.async_copy(f_hbm.at[idx_v.at[j]], rows_v, sem).wait()
            pltpu.sync_copy(rows_v,
                            out_hbm.at[pl.ds(wid * 256 + j * 128, 128)])

    return k(features, idx2)


def kernel(queries, features):
    sim, cmax3 = _sim_chunkmax(queries, features)
    cmax = jnp.transpose(cmax3, (1, 0, 2)).reshape(QN, NCHUNK)
    cflat = _chunk_topk(cmax)
    cand = _sc_gather_cand(sim.reshape(QN * NCHUNK, CHUNK),
                           cflat.reshape(64, 128))
    fidx = _final_topk(cand.reshape(QN, TK * CHUNK), cflat)
    sel = _sc_gather_feats(features, fidx.reshape(64, 128))
    subgraph_feats = jnp.concatenate([queries, sel], axis=0)
    src = jnp.repeat(jnp.arange(QN, dtype=jnp.int64), TK)
    dst = QN + jnp.tile(jnp.arange(TK, dtype=jnp.int64), QN)
    edge_index = jnp.stack([src, dst], axis=0)
    return subgraph_feats, edge_index
